# baseline (speedup 1.0000x reference)
import jax
import jax.numpy as jnp
from jax.experimental import pallas as pl

N = 4096
IN_F = 128
OUT_F = 128
ALPHA = 0.2
BM = 256
LOG2E = 1.4426950408889634


def _proj_kernel(x_ref, w_ref, a1_ref, a2_ref,
                 hb_ref, srcs_ref, dsts_ref, dmax_ref, meanh_ref):
    h = jnp.dot(x_ref[...], w_ref[...], preferred_element_type=jnp.float32)
    hb_ref[...] = h.astype(jnp.bfloat16)
    meanh_ref[...] = jnp.mean(h, axis=0, keepdims=True)
    srcs_ref[...] = jnp.dot(h, a1_ref[...], preferred_element_type=jnp.float32) * LOG2E
    dsts = jnp.dot(h, a2_ref[...], preferred_element_type=jnp.float32) * LOG2E
    dsts_ref[...] = dsts
    dmax_ref[...] = jnp.max(dsts).reshape(1, 1)


def _attn_kernel(adj_ref, hb_ref, srcs_ref, dstts_ref, dmax_ref, meanh_ref,
                 out_ref):
    srcs = srcs_ref[...]
    t = srcs + dmax_ref[0, 0]
    m = jnp.maximum(t, ALPHA * t)
    l0 = srcs + dstts_ref[...]
    lk = jnp.maximum(l0, ALPHA * l0)
    e = jnp.exp2(lk - m) * adj_ref[...]
    s = jnp.sum(e, axis=1, keepdims=True)
    hp = jnp.dot(e.astype(jnp.bfloat16), hb_ref[...],
                 preferred_element_type=jnp.float32)
    s_safe = jnp.where(s > 0, s, 1.0)
    hp = jnp.where(s > 0, hp / s_safe, meanh_ref[...])
    out_ref[...] = jnp.where(hp > 0, hp, jnp.exp(jnp.minimum(hp, 0.0)) - 1.0)


@jax.jit
def kernel(input, adj, W, a):
    a1 = a[:OUT_F].reshape(IN_F, 1)
    a2 = a[OUT_F:].reshape(IN_F, 1)
    hb, srcs, dsts, dmax, meanh = pl.pallas_call(
        _proj_kernel,
        out_shape=(
            jax.ShapeDtypeStruct((N, OUT_F), jnp.bfloat16),
            jax.ShapeDtypeStruct((N, 1), jnp.float32),
            jax.ShapeDtypeStruct((N, 1), jnp.float32),
            jax.ShapeDtypeStruct((1, 1), jnp.float32),
            jax.ShapeDtypeStruct((1, OUT_F), jnp.float32),
        ),
    )(input, W, a1, a2)
    dstts = dsts.reshape(1, N)

    out = pl.pallas_call(
        _attn_kernel,
        grid=(N // BM,),
        in_specs=[
            pl.BlockSpec((BM, N), lambda i: (i, 0)),
            pl.BlockSpec((N, OUT_F), lambda i: (0, 0)),
            pl.BlockSpec((BM, 1), lambda i: (i, 0)),
            pl.BlockSpec((1, N), lambda i: (0, 0)),
            pl.BlockSpec((1, 1), lambda i: (0, 0)),
            pl.BlockSpec((1, OUT_F), lambda i: (0, 0)),
        ],
        out_specs=pl.BlockSpec((BM, OUT_F), lambda i: (i, 0)),
        out_shape=jax.ShapeDtypeStruct((N, OUT_F), jnp.float32),
    )(adj, hb, srcs, dstts, dmax, meanh)
    return out


# speedup vs baseline: 1.1873x; 1.1873x over previous
import jax
import jax.numpy as jnp
from jax.experimental import pallas as pl

N = 4096
IN_F = 128
OUT_F = 128
ALPHA = 0.2
BM = 512
HA = 256
LOG2E = 1.4426950408889634


def _proj_kernel(x_ref, w_ref, a1_ref, a2_ref,
                 hb1_ref, srcs_ref, dsts_ref, dmax_ref, meanh_ref):
    h = jnp.dot(x_ref[...], w_ref[...], preferred_element_type=jnp.float32)
    hb1_ref[:, :OUT_F] = h.astype(jnp.bfloat16)
    hb1_ref[:, OUT_F:OUT_F + 1] = jnp.ones((N, 1), jnp.bfloat16)
    hb1_ref[:, OUT_F + 1:] = jnp.zeros((N, HA - OUT_F - 1), jnp.bfloat16)
    meanh_ref[...] = jnp.mean(h, axis=0, keepdims=True)
    srcs_ref[...] = jnp.dot(h, a1_ref[...], preferred_element_type=jnp.float32) * LOG2E
    dsts = jnp.dot(h, a2_ref[...], preferred_element_type=jnp.float32) * LOG2E
    dsts_ref[...] = dsts
    dmax_ref[...] = jnp.max(dsts).reshape(1, 1)


def _attn_kernel(adj_ref, hb1_ref, srcs_ref, dstts_ref, dmax_ref, meanh_ref,
                 out_ref):
    srcs = srcs_ref[...]
    t = srcs + dmax_ref[0, 0]
    m = jnp.maximum(t, ALPHA * t)
    sa = srcs - m
    sb = ALPHA * srcs - m
    dstts = dstts_ref[...]
    dstts2 = ALPHA * dstts
    e = jnp.exp2(jnp.maximum(sa + dstts, sb + dstts2)) * adj_ref[...]
    hp1 = jnp.dot(e.astype(jnp.bfloat16), hb1_ref[...],
                  preferred_element_type=jnp.float32)
    s = hp1[:, OUT_F:OUT_F + 1]
    hp = hp1[:, :OUT_F]
    s_safe = jnp.where(s > 0, s, 1.0)
    hp = jnp.where(s > 0, hp / s_safe, meanh_ref[...])
    out_ref[...] = jnp.where(hp > 0, hp, jnp.exp(jnp.minimum(hp, 0.0)) - 1.0)


@jax.jit
def kernel(input, adj, W, a):
    a1 = a[:OUT_F].reshape(IN_F, 1)
    a2 = a[OUT_F:].reshape(IN_F, 1)
    hb1, srcs, dsts, dmax, meanh = pl.pallas_call(
        _proj_kernel,
        out_shape=(
            jax.ShapeDtypeStruct((N, HA), jnp.bfloat16),
            jax.ShapeDtypeStruct((N, 1), jnp.float32),
            jax.ShapeDtypeStruct((N, 1), jnp.float32),
            jax.ShapeDtypeStruct((1, 1), jnp.float32),
            jax.ShapeDtypeStruct((1, OUT_F), jnp.float32),
        ),
    )(input, W, a1, a2)
    dstts = dsts.reshape(1, N)

    out = pl.pallas_call(
        _attn_kernel,
        grid=(N // BM,),
        in_specs=[
            pl.BlockSpec((BM, N), lambda i: (i, 0)),
            pl.BlockSpec((N, HA), lambda i: (0, 0)),
            pl.BlockSpec((BM, 1), lambda i: (i, 0)),
            pl.BlockSpec((1, N), lambda i: (0, 0)),
            pl.BlockSpec((1, 1), lambda i: (0, 0)),
            pl.BlockSpec((1, OUT_F), lambda i: (0, 0)),
        ],
        out_specs=pl.BlockSpec((BM, OUT_F), lambda i: (i, 0)),
        out_shape=jax.ShapeDtypeStruct((N, OUT_F), jnp.float32),
    )(adj, hb1, srcs, dstts, dmax, meanh)
    return out


# speedup vs baseline: 1.5839x; 1.3339x over previous
import jax
import jax.numpy as jnp
from jax.experimental import pallas as pl
from jax.experimental.pallas import tpu as pltpu

N = 4096
IN_F = 128
OUT_F = 128
ALPHA = 0.2
BM = 512
HA = 256
LOG2E = 1.4426950408889634


def _gat_kernel(adj_ref, x_ref, w_ref, a1_ref, a2r_ref, out_ref,
                hb1_s, srcs_s, dstt_s, dmax_s, meanh_s):
    i = pl.program_id(0)

    @pl.when(i == 0)
    def _init():
        h = jnp.dot(x_ref[...], w_ref[...], preferred_element_type=jnp.float32)
        hb1_s[:, :OUT_F] = h.astype(jnp.bfloat16)
        hb1_s[:, OUT_F:OUT_F + 1] = jnp.ones((N, 1), jnp.bfloat16)
        hb1_s[:, OUT_F + 1:] = jnp.zeros((N, HA - OUT_F - 1), jnp.bfloat16)
        meanh_s[...] = jnp.mean(h, axis=0, keepdims=True)
        srcs_s[...] = jnp.dot(h, a1_ref[...],
                              preferred_element_type=jnp.float32) * LOG2E
        dstt = jax.lax.dot_general(
            a2r_ref[...], h, (((1,), (1,)), ((), ())),
            preferred_element_type=jnp.float32) * LOG2E
        dstt_s[...] = dstt
        dmax_s[...] = jnp.max(dstt).reshape(1, 1)

    srcs = srcs_s[pl.ds(i * BM, BM), :]
    t = srcs + dmax_s[0, 0]
    m = jnp.maximum(t, ALPHA * t)
    sa = srcs - m
    sb = ALPHA * srcs - m
    dstts = dstt_s[...]
    dstts2 = ALPHA * dstts
    e = jnp.exp2(jnp.maximum(sa + dstts, sb + dstts2)) * adj_ref[...]
    hp1 = jnp.dot(e.astype(jnp.bfloat16), hb1_s[...],
                  preferred_element_type=jnp.float32)
    s = hp1[:, OUT_F:OUT_F + 1]
    hp = hp1[:, :OUT_F]
    s_safe = jnp.where(s > 0, s, 1.0)
    hp = jnp.where(s > 0, hp / s_safe, meanh_s[...])
    out_ref[...] = jnp.where(hp > 0, hp, jnp.exp(jnp.minimum(hp, 0.0)) - 1.0)


@jax.jit
def kernel(input, adj, W, a):
    a1 = a[:OUT_F].reshape(IN_F, 1)
    a2r = a[OUT_F:].reshape(1, IN_F)
    out = pl.pallas_call(
        _gat_kernel,
        grid=(N // BM,),
        in_specs=[
            pl.BlockSpec((BM, N), lambda i: (i, 0)),
            pl.BlockSpec((N, IN_F), lambda i: (0, 0)),
            pl.BlockSpec((IN_F, OUT_F), lambda i: (0, 0)),
            pl.BlockSpec((IN_F, 1), lambda i: (0, 0)),
            pl.BlockSpec((1, IN_F), lambda i: (0, 0)),
        ],
        out_specs=pl.BlockSpec((BM, OUT_F), lambda i: (i, 0)),
        out_shape=jax.ShapeDtypeStruct((N, OUT_F), jnp.float32),
        scratch_shapes=[
            pltpu.VMEM((N, HA), jnp.bfloat16),
            pltpu.VMEM((N, 1), jnp.float32),
            pltpu.VMEM((1, N), jnp.float32),
            pltpu.VMEM((1, 1), jnp.float32),
            pltpu.VMEM((1, OUT_F), jnp.float32),
        ],
    )(adj, input, W, a1, a2r)
    return out
